# baseline (speedup 1.0000x reference)
import functools

import jax
import jax.numpy as jnp
from jax import lax
from jax.experimental import pallas as pl
from jax.experimental.pallas import tpu as pltpu
from jax.experimental.pallas import tpu_sc as plsc

NCOIL = 8
NPE = 256
NRO_OS = 2048
NPTS = 4096
OVERSAMP = 4.0

B = NCOIL * NPE
N = NRO_OS
P = NPTS
L = 16
NC = 2
NS = 16
NW = NC * NS
ROWS_PER_TILE = B // NW
RG = 8
GROUPS = ROWS_PER_TILE // RG
QS = 8
NOB = 4
QW = P // QS
QCHUNKS = QW // L


def _body(c_hbm, kr_hbm, ki_hbm, or_hbm, oi_hbm,
          c_v, rr0, rr1, ri0, ri1,
          our0, our1, our2, our3, oui0, oui1, oui2, oui3,
          s_c, s_ir0, s_ir1, s_ii0, s_ii1,
          s_or0, s_or1, s_or2, s_or3, s_oi0, s_oi1, s_oi2, s_oi3):
    wid = lax.axis_index("s") * NC + lax.axis_index("c")
    row0 = wid * ROWS_PER_TILE

    rr = (rr0, rr1)
    ri = (ri0, ri1)
    our = (our0, our1, our2, our3)
    oui = (oui0, oui1, oui2, oui3)
    s_ir = (s_ir0, s_ir1)
    s_ii = (s_ii0, s_ii1)
    s_or = (s_or0, s_or1, s_or2, s_or3)
    s_oi = (s_oi0, s_oi1, s_oi2, s_oi3)

    def in_cps_buf(g, b):
        base = row0 + g * RG
        return (
            pltpu.make_async_copy(kr_hbm.at[pl.ds(base, RG)], rr[b], s_ir[b]),
            pltpu.make_async_copy(ki_hbm.at[pl.ds(base, RG)], ri[b], s_ii[b]),
        )

    def in_cps(g):
        return in_cps_buf(g, g & 1)

    def out_cps(g, q):
        b = q % NOB
        base = row0 + g * RG
        cols = pl.ds(q * QW, QW)
        return (
            pltpu.make_async_copy(our[b], or_hbm.at[pl.ds(base, RG), cols], s_or[b]),
            pltpu.make_async_copy(oui[b], oi_hbm.at[pl.ds(base, RG), cols], s_oi[b]),
        )

    c_cp = pltpu.make_async_copy(c_hbm, c_v, s_c)
    c_cp.start()
    for cp in in_cps(0):
        cp.start()
    for cp in in_cps(1):
        cp.start()
    c_cp.wait()

    def do_group(g, gb):
        for cp in in_cps_buf(g, gb):
            cp.wait()
        rr_b, ri_b = rr[gb], ri[gb]
        for q in range(QS):
            if q >= NOB:
                for cp in out_cps(g, q):
                    cp.wait()
            else:
                @pl.when(g >= 1)
                def _():
                    for cp in out_cps(g, q):
                        cp.wait()
            our_b, oui_b = our[q % NOB], oui[q % NOB]
            q0 = q * QW

            @plsc.parallel_loop(0, QCHUNKS, 1, unroll=1)
            def chunk(i):
                s = i * L
                cq = c_v[pl.ds(q0 + s, L)]
                kx = cq * OVERSAMP
                i0 = kx.astype(jnp.int32)
                w1 = kx - i0.astype(jnp.float32)
                w0 = 1.0 - w1
                idx0 = lax.bitwise_and(i0, N - 1)
                idx1 = lax.bitwise_and(i0 + 1, N - 1)
                for r in range(RG):
                    rv = jnp.full((L,), r, jnp.int32)
                    g0 = plsc.load_gather(rr_b, [rv, idx0])
                    g1 = plsc.load_gather(rr_b, [rv, idx1])
                    our_b[r, pl.ds(s, L)] = w0 * g0 + w1 * g1
                    h0 = plsc.load_gather(ri_b, [rv, idx0])
                    h1 = plsc.load_gather(ri_b, [rv, idx1])
                    oui_b[r, pl.ds(s, L)] = w0 * h0 + w1 * h1

            for cp in out_cps(g, q):
                cp.start()

        @pl.when(g + 2 < GROUPS)
        def _():
            for cp in in_cps_buf(g + 2, gb):
                cp.start()

    def pair(g2, carry):
        do_group(g2 * 2, 0)
        do_group(g2 * 2 + 1, 1)
        return carry

    lax.fori_loop(0, GROUPS // 2, pair, 0)

    for q in range(QS - NOB, QS):
        for cp in out_cps(GROUPS - 1, q):
            cp.wait()


@jax.jit
def _interp(c_flat, kr, ki):
    mesh = plsc.VectorSubcoreMesh(
        core_axis_name="c", subcore_axis_name="s", num_cores=NC, num_subcores=NS
    )
    f = pl.kernel(
        _body,
        out_type=(
            jax.ShapeDtypeStruct((B, P), jnp.float32),
            jax.ShapeDtypeStruct((B, P), jnp.float32),
        ),
        mesh=mesh,
        scratch_types=[
            pltpu.VMEM((P,), jnp.float32),
            pltpu.VMEM((RG, N), jnp.float32),
            pltpu.VMEM((RG, N), jnp.float32),
            pltpu.VMEM((RG, N), jnp.float32),
            pltpu.VMEM((RG, N), jnp.float32),
            pltpu.VMEM((RG, QW), jnp.float32),
            pltpu.VMEM((RG, QW), jnp.float32),
            pltpu.VMEM((RG, QW), jnp.float32),
            pltpu.VMEM((RG, QW), jnp.float32),
            pltpu.VMEM((RG, QW), jnp.float32),
            pltpu.VMEM((RG, QW), jnp.float32),
            pltpu.VMEM((RG, QW), jnp.float32),
            pltpu.VMEM((RG, QW), jnp.float32),
        ] + [pltpu.SemaphoreType.DMA] * 13,
        compiler_params=pltpu.CompilerParams(
            use_tc_tiling_on_sc=True, needs_layout_passes=False
        ),
    )
    return f(c_flat, kr, ki)


def kernel(c, ksp_real, ksp_imag):
    batch_shape = ksp_real.shape[:-1]
    pts_shape = c.shape[:-1]
    c_flat = c.reshape(-1)
    kr = ksp_real.reshape(-1, N)
    ki = ksp_imag.reshape(-1, N)
    out_r, out_i = _interp(c_flat, kr, ki)
    out = lax.complex(out_r, out_i)
    return out.reshape(batch_shape + pts_shape)


# speedup vs baseline: 1.0197x; 1.0197x over previous
import jax
import jax.numpy as jnp
from jax import lax
from jax.experimental import pallas as pl
from jax.experimental.pallas import tpu as pltpu
from jax.experimental.pallas import tpu_sc as plsc

NCOIL = 8
NPE = 256
NRO_OS = 2048
NPTS = 4096
OVERSAMP = 4.0

B = NCOIL * NPE
N = NRO_OS
P = NPTS
L = 16
NC = 2
NS = 16
NW = NC * NS
ROWS_PER_TILE = B // NW
RG = 4
GROUPS = ROWS_PER_TILE // RG
CHUNKS = P // L
UNROLL = 2


def _body(c_hbm, kr_hbm, ki_hbm, or_hbm, oi_hbm,
          c_v, rr0, rr1, ri0, ri1, our0, our1, oui0, oui1,
          s_c, s_ir0, s_ir1, s_ii0, s_ii1, s_or0, s_or1, s_oi0, s_oi1):
    wid = lax.axis_index("s") * NC + lax.axis_index("c")
    row0 = wid * ROWS_PER_TILE

    rr = (rr0, rr1)
    ri = (ri0, ri1)
    our = (our0, our1)
    oui = (oui0, oui1)
    s_ir = (s_ir0, s_ir1)
    s_ii = (s_ii0, s_ii1)
    s_or = (s_or0, s_or1)
    s_oi = (s_oi0, s_oi1)

    def in_cps(g):
        b = g & 1
        base = row0 + g * RG
        return (
            pltpu.make_async_copy(kr_hbm.at[pl.ds(base, RG)], rr[b], s_ir[b]),
            pltpu.make_async_copy(ki_hbm.at[pl.ds(base, RG)], ri[b], s_ii[b]),
        )

    def out_cps(g):
        b = g & 1
        base = row0 + g * RG
        return (
            pltpu.make_async_copy(our[b], or_hbm.at[pl.ds(base, RG)], s_or[b]),
            pltpu.make_async_copy(oui[b], oi_hbm.at[pl.ds(base, RG)], s_oi[b]),
        )

    c_cp = pltpu.make_async_copy(c_hbm, c_v, s_c)
    c_cp.start()
    for cp in in_cps(0):
        cp.start()
    c_cp.wait()

    out_pending = {}
    for g in range(GROUPS):
        b = g & 1
        if g + 1 < GROUPS:
            for cp in in_cps(g + 1):
                cp.start()
        for cp in in_cps(g):
            cp.wait()
        if g >= 2:
            for cp in out_pending.pop(g - 2):
                cp.wait()
        rr_b, ri_b, our_b, oui_b = rr[b], ri[b], our[b], oui[b]

        @plsc.parallel_loop(0, CHUNKS, 1, unroll=UNROLL)
        def chunk(i):
            s = i * L
            cq = c_v[pl.ds(s, L)]
            kx = cq * OVERSAMP
            i0 = kx.astype(jnp.int32)
            w1 = kx - i0.astype(jnp.float32)
            w0 = 1.0 - w1
            idx0 = lax.bitwise_and(i0, N - 1)
            idx1 = lax.bitwise_and(i0 + 1, N - 1)
            for r in range(RG):
                rv = jnp.full((L,), r, jnp.int32)
                g0 = plsc.load_gather(rr_b, [rv, idx0])
                g1 = plsc.load_gather(rr_b, [rv, idx1])
                our_b[r, pl.ds(s, L)] = w0 * g0 + w1 * g1
                h0 = plsc.load_gather(ri_b, [rv, idx0])
                h1 = plsc.load_gather(ri_b, [rv, idx1])
                oui_b[r, pl.ds(s, L)] = w0 * h0 + w1 * h1

        cps = out_cps(g)
        for cp in cps:
            cp.start()
        out_pending[g] = cps

    for g in sorted(out_pending):
        for cp in out_pending[g]:
            cp.wait()


@jax.jit
def _interp(c_flat, kr, ki):
    mesh = plsc.VectorSubcoreMesh(
        core_axis_name="c", subcore_axis_name="s", num_cores=NC, num_subcores=NS
    )
    f = pl.kernel(
        _body,
        out_type=(
            jax.ShapeDtypeStruct((B, P), jnp.float32),
            jax.ShapeDtypeStruct((B, P), jnp.float32),
        ),
        mesh=mesh,
        scratch_types=[
            pltpu.VMEM((P,), jnp.float32),
            pltpu.VMEM((RG, N), jnp.float32),
            pltpu.VMEM((RG, N), jnp.float32),
            pltpu.VMEM((RG, N), jnp.float32),
            pltpu.VMEM((RG, N), jnp.float32),
            pltpu.VMEM((RG, P), jnp.float32),
            pltpu.VMEM((RG, P), jnp.float32),
            pltpu.VMEM((RG, P), jnp.float32),
            pltpu.VMEM((RG, P), jnp.float32),
        ] + [pltpu.SemaphoreType.DMA] * 9,
        compiler_params=pltpu.CompilerParams(
            use_tc_tiling_on_sc=True, needs_layout_passes=False
        ),
    )
    return f(c_flat, kr, ki)


def kernel(c, ksp_real, ksp_imag):
    batch_shape = ksp_real.shape[:-1]
    pts_shape = c.shape[:-1]
    c_flat = c.reshape(-1)
    kr = ksp_real.reshape(-1, N)
    ki = ksp_imag.reshape(-1, N)
    out_r, out_i = _interp(c_flat, kr, ki)
    out = lax.complex(out_r, out_i)
    return out.reshape(batch_shape + pts_shape)
